# baseline (speedup 1.0000x reference)
import functools

import numpy as np
import jax
import jax.numpy as jnp
from jax import lax
from jax.experimental import pallas as pl
from jax.experimental.pallas import tpu as pltpu
from jax.experimental.pallas import tpu_sc as plsc

_SYNC_COST = 10.0
_EPS = 1e-07
_T_TRIM = 500
_T_FULL = 600
_N_TRIALS = 16
_N_NEURONS = 10000
_N_SAMPLES = 50
_NS_PAD = 64
_N_E = 8000

_KP = 512
_TB = 128
_MP = 4096


def _sampling_plan():
    rng = np.random.default_rng(0)
    trials = rng.integers(0, _N_TRIALS, size=_N_SAMPLES)
    counts = rng.normal(70.0, 30.0, size=_N_SAMPLES).astype(np.int32)
    counts = np.clip(counts, 15, _N_E)
    shuffled = rng.permutation(_N_E)
    prev = 0
    slots, samp = [], []
    for i in range(_N_SAMPLES):
        n = int(counts[i])
        if prev + n > shuffled.shape[0]:
            shuffled = rng.permutation(_N_E)
            prev = 0
        slots.append(shuffled[prev:prev + n].copy())
        samp.append(np.full(n, i, np.int32))
        prev += n
    return trials, np.concatenate(slots).astype(np.int32), np.concatenate(samp)


_TRIALS, _SLOT, _SAMP = _sampling_plan()
_M = _SLOT.shape[0]

_SLOT_PAD = np.zeros(_MP, np.int32)
_SLOT_PAD[:_M] = _SLOT

_ONEHOT = np.zeros((_MP, _NS_PAD), np.float32)
_ONEHOT[np.arange(_M), _SAMP] = 1.0

_USED = np.asarray(sorted(set(_TRIALS.tolist())), np.int32)
_NT_USED = _USED.shape[0]
_GAP_THRESH = [int(m) - i for i, m in
               enumerate(sorted(set(range(_N_TRIALS)) - set(_TRIALS.tolist())))]

_TRIALMASK = np.zeros((_NT_USED, 1, _NS_PAD), np.float32)
for _k, _t in enumerate(_USED.tolist()):
    _TRIALMASK[_k, 0, np.arange(_N_SAMPLES)[_TRIALS == _t]] = 1.0

_MASK50 = np.zeros((1, _NS_PAD), np.float32)
_MASK50[0, :_N_SAMPLES] = 1.0


def _bin_plan():
    bs = np.logspace(-3, 0, 20)
    bs = bs[bs < 0.25]
    plan, roff = [], 0
    for bw in bs:
        b = int(np.round(bw * 1000))
        nb = _T_TRIM // b
        plan.append((b, nb, roff))
        roff += nb
    return plan, roff


_BINS, _NROWS = _bin_plan()

_BALL = np.zeros((_NROWS, _KP), np.float32)
for _b, _nb, _roff in _BINS:
    for _r in range(_nb):
        _BALL[_roff + _r, _r * _b:(_r + 1) * _b] = 1.0

_ZROWS = np.zeros((10240, _NS_PAD), np.float32)

_MS = _MP // 16
_NR_PAD = 10240
_RS = _NR_PAD // 16


def _sc_membership(node_id_e, slot, onehot, zrows):
    mesh = plsc.VectorSubcoreMesh(core_axis_name="c", subcore_axis_name="s")

    @functools.partial(
        pl.kernel,
        mesh=mesh,
        out_type=jax.ShapeDtypeStruct((_NR_PAD, _NS_PAD), jnp.float32),
        scratch_types=[
            pltpu.VMEM((_MS,), jnp.int32),
            pltpu.VMEM((_MS // 128, 128), jnp.int32),
            pltpu.VMEM((_MS, _NS_PAD), jnp.float32),
            pltpu.VMEM_SHARED((_NR_PAD, _NS_PAD), jnp.float32),
            pltpu.SemaphoreType.DMA,
            pltpu.SemaphoreType.DMA,
            pltpu.SemaphoreType.DMA,
        ],
    )
    def body(nid_h, slot_h, oh_h, z_h, out_h, slotv, nidv, ohv, bsh,
             sem, zsem, osem):
        cid = lax.axis_index("c")
        sid = lax.axis_index("s")

        @pl.when(cid == 0)
        def _():
            zcp = pltpu.make_async_copy(z_h.at[pl.ds(sid * _RS, _RS)],
                                        bsh.at[pl.ds(sid * _RS, _RS)], zsem)
            zcp.start()
            ocp = pltpu.make_async_copy(oh_h.at[pl.ds(sid * _MS, _MS)], ohv,
                                        osem)
            ocp.start()
            pltpu.sync_copy(slot_h.at[pl.ds(sid * _MS, _MS)], slotv)
            for j in range(_MS // 128):
                pltpu.make_async_copy(
                    nid_h.at[slotv.at[pl.ds(j * 128, 128)]], nidv.at[j], sem
                ).start()
            for j in range(_MS // 128):
                pltpu.make_async_copy(
                    nid_h.at[slotv.at[pl.ds(j * 128, 128)]], nidv.at[j], sem
                ).wait()
            zcp.wait()
            ocp.wait()
            plsc.subcore_barrier()
            for j in range(_MS // 128):
                pltpu.sync_copy(ohv.at[pl.ds(j * 128, 128)],
                                bsh.at[nidv.at[j]], add=True)
            plsc.subcore_barrier()
            pltpu.sync_copy(bsh.at[pl.ds(sid * _RS, _RS)],
                            out_h.at[pl.ds(sid * _RS, _RS)])

    return body(node_id_e, slot, onehot, zrows)


def _tc_body(spk_ref, b_ref, tm_ref, m_ref, e_ref, o_ref, selv):
    tb = pl.program_id(0)
    tr = pl.program_id(1)
    res = jnp.dot(spk_ref[0], b_ref[0:_N_NEURONS, :],
                  preferred_element_type=jnp.float32)
    res = res * tm_ref[0]

    @pl.when(tr == 0)
    def _():
        selv[pl.ds(tb * _TB, _TB), :] = res

    @pl.when(tr != 0)
    def _():
        selv[pl.ds(tb * _TB, _TB), :] = selv[pl.ds(tb * _TB, _TB), :] + res

    @pl.when((tb == _KP // _TB - 1) & (tr == _NT_USED - 1))
    def _():
        sel = selv[...]
        msk = m_ref[...]
        acc = jnp.float32(0.0)
        for bi, (bs, nb, _roff) in enumerate(_BINS):
            blk = jnp.sum(sel[0:nb * bs, :].reshape(nb, bs, _NS_PAD), axis=1)
            mean = jnp.mean(blk, axis=0, keepdims=True)
            var = jnp.mean((blk - mean) ** 2, axis=0, keepdims=True)
            fano = var / jnp.maximum(mean, _EPS)
            fm = jnp.sum(fano * msk) / jnp.float32(_N_SAMPLES)
            d = e_ref[0, bi] - fm
            acc = acc + d * d
        o_ref[0, 0] = _SYNC_COST * acc / jnp.float32(len(_BINS))


def _used_trial(tr):
    t = tr
    for th in _GAP_THRESH:
        t = t + (tr >= th).astype(jnp.int32)
    return t


def _tc_main(spikes, bmat, trialmask, msk, exp2):
    return pl.pallas_call(
        _tc_body,
        grid=(_KP // _TB, _NT_USED),
        in_specs=[
            pl.BlockSpec((1, _TB, _N_NEURONS),
                         lambda tb, tr: (_used_trial(tr), tb, 0)),
            pl.BlockSpec((_NR_PAD, _NS_PAD), lambda tb, tr: (0, 0)),
            pl.BlockSpec((1, 1, _NS_PAD), lambda tb, tr: (tr, 0, 0)),
            pl.BlockSpec((1, _NS_PAD), lambda tb, tr: (0, 0)),
            pl.BlockSpec(memory_space=pltpu.SMEM),
        ],
        out_specs=pl.BlockSpec(memory_space=pltpu.SMEM),
        out_shape=jax.ShapeDtypeStruct((1, 1), jnp.float32),
        scratch_shapes=[pltpu.VMEM((_KP, _NS_PAD), jnp.float32)],
    )(spikes, bmat, trialmask, msk, exp2)


def kernel(spikes, node_id_e, experimental_fanos_mean):
    spikes = spikes.astype(jnp.float32)
    slot = jnp.asarray(_SLOT_PAD)
    onehot = jnp.asarray(_ONEHOT)
    zrows = jnp.asarray(_ZROWS)
    bmat = _sc_membership(node_id_e.astype(jnp.int32), slot, onehot, zrows)
    exp2 = experimental_fanos_mean.astype(jnp.float32).reshape(1, len(_BINS))
    res = _tc_main(spikes, bmat, jnp.asarray(_TRIALMASK),
                   jnp.asarray(_MASK50), exp2)
    return res[0, 0]


# speedup vs baseline: 1.1109x; 1.1109x over previous
import functools

import numpy as np
import jax
import jax.numpy as jnp
from jax import lax
from jax.experimental import pallas as pl
from jax.experimental.pallas import tpu as pltpu
from jax.experimental.pallas import tpu_sc as plsc

_SYNC_COST = 10.0
_EPS = 1e-07
_T_TRIM = 500
_T_FULL = 600
_N_TRIALS = 16
_N_NEURONS = 10000
_N_SAMPLES = 50
_NS_PAD = 64
_N_E = 8000

_KP = 512
_TB = 256
_MP = 4096


def _sampling_plan():
    rng = np.random.default_rng(0)
    trials = rng.integers(0, _N_TRIALS, size=_N_SAMPLES)
    counts = rng.normal(70.0, 30.0, size=_N_SAMPLES).astype(np.int32)
    counts = np.clip(counts, 15, _N_E)
    shuffled = rng.permutation(_N_E)
    prev = 0
    slots, samp = [], []
    for i in range(_N_SAMPLES):
        n = int(counts[i])
        if prev + n > shuffled.shape[0]:
            shuffled = rng.permutation(_N_E)
            prev = 0
        slots.append(shuffled[prev:prev + n].copy())
        samp.append(np.full(n, i, np.int32))
        prev += n
    return trials, np.concatenate(slots).astype(np.int32), np.concatenate(samp)


_TRIALS, _SLOT, _SAMP = _sampling_plan()
_M = _SLOT.shape[0]

_SLOT_PAD = np.zeros(_MP, np.int32)
_SLOT_PAD[:_M] = _SLOT

_ONEHOT = np.zeros((_MP, _NS_PAD), np.float32)
_ONEHOT[np.arange(_M), _SAMP] = 1.0

_USED = np.asarray(sorted(set(_TRIALS.tolist())), np.int32)
_NT_USED = _USED.shape[0]
_GAP_THRESH = [int(m) - i for i, m in
               enumerate(sorted(set(range(_N_TRIALS)) - set(_TRIALS.tolist())))]

_TRIALMASK = np.zeros((_NT_USED, 1, _NS_PAD), np.float32)
for _k, _t in enumerate(_USED.tolist()):
    _TRIALMASK[_k, 0, np.arange(_N_SAMPLES)[_TRIALS == _t]] = 1.0

_MASK50 = np.zeros((1, _NS_PAD), np.float32)
_MASK50[0, :_N_SAMPLES] = 1.0


def _bin_plan():
    bs = np.logspace(-3, 0, 20)
    bs = bs[bs < 0.25]
    plan, roff = [], 0
    for bw in bs:
        b = int(np.round(bw * 1000))
        nb = _T_TRIM // b
        plan.append((b, nb, roff))
        roff += nb
    return plan, roff


_BINS, _NROWS = _bin_plan()

_ZROWS = np.zeros((10240, _NS_PAD), np.float32)

_MS = _MP // 16
_NR_PAD = 10240
_RS = _NR_PAD // 16


def _sc_membership(node_id_e, slot, onehot, zrows):
    mesh = plsc.VectorSubcoreMesh(core_axis_name="c", subcore_axis_name="s")

    @functools.partial(
        pl.kernel,
        mesh=mesh,
        out_type=jax.ShapeDtypeStruct((_NR_PAD, _NS_PAD), jnp.float32),
        scratch_types=[
            pltpu.VMEM((_MS,), jnp.int32),
            pltpu.VMEM((_MS // 128, 128), jnp.int32),
            pltpu.VMEM((_MS, _NS_PAD), jnp.float32),
            pltpu.VMEM_SHARED((_NR_PAD, _NS_PAD), jnp.float32),
            pltpu.SemaphoreType.DMA,
            pltpu.SemaphoreType.DMA,
            pltpu.SemaphoreType.DMA,
        ],
    )
    def body(nid_h, slot_h, oh_h, z_h, out_h, slotv, nidv, ohv, bsh,
             sem, zsem, osem):
        cid = lax.axis_index("c")
        sid = lax.axis_index("s")

        @pl.when(cid == 0)
        def _():
            zcp = pltpu.make_async_copy(z_h.at[pl.ds(sid * _RS, _RS)],
                                        bsh.at[pl.ds(sid * _RS, _RS)], zsem)
            zcp.start()
            ocp = pltpu.make_async_copy(oh_h.at[pl.ds(sid * _MS, _MS)], ohv,
                                        osem)
            ocp.start()
            pltpu.sync_copy(slot_h.at[pl.ds(sid * _MS, _MS)], slotv)
            for j in range(_MS // 128):
                pltpu.make_async_copy(
                    nid_h.at[slotv.at[pl.ds(j * 128, 128)]], nidv.at[j], sem
                ).start()
            for j in range(_MS // 128):
                pltpu.make_async_copy(
                    nid_h.at[slotv.at[pl.ds(j * 128, 128)]], nidv.at[j], sem
                ).wait()
            zcp.wait()
            ocp.wait()
            plsc.subcore_barrier()
            for j in range(_MS // 128):
                pltpu.sync_copy(ohv.at[pl.ds(j * 128, 128)],
                                bsh.at[nidv.at[j]], add=True)
            plsc.subcore_barrier()
            pltpu.sync_copy(bsh.at[pl.ds(sid * _RS, _RS)],
                            out_h.at[pl.ds(sid * _RS, _RS)])

    return body(node_id_e, slot, onehot, zrows)


def _tc_body(spk_ref, b_ref, tm_ref, m_ref, e_ref, o_ref, selv):
    tb = pl.program_id(0)
    tr = pl.program_id(1)
    res = jnp.dot(spk_ref[0], b_ref[0:_N_NEURONS, :],
                  preferred_element_type=jnp.float32)
    res = res * tm_ref[0]

    @pl.when(tr == 0)
    def _():
        selv[pl.ds(tb * _TB, _TB), :] = res

    @pl.when(tr != 0)
    def _():
        selv[pl.ds(tb * _TB, _TB), :] = selv[pl.ds(tb * _TB, _TB), :] + res

    @pl.when((tb == _KP // _TB - 1) & (tr == _NT_USED - 1))
    def _():
        sel = selv[...]
        msk = m_ref[...]
        acc = jnp.float32(0.0)
        for bi, (bs, nb, _roff) in enumerate(_BINS):
            blk = jnp.sum(sel[0:nb * bs, :].reshape(nb, bs, _NS_PAD), axis=1)
            mean = jnp.mean(blk, axis=0, keepdims=True)
            var = jnp.mean((blk - mean) ** 2, axis=0, keepdims=True)
            fano = var / jnp.maximum(mean, _EPS)
            fm = jnp.sum(fano * msk) / jnp.float32(_N_SAMPLES)
            d = e_ref[0, bi] - fm
            acc = acc + d * d
        o_ref[0, 0] = _SYNC_COST * acc / jnp.float32(len(_BINS))


def _used_trial(tr):
    t = tr
    for th in _GAP_THRESH:
        t = t + (tr >= th).astype(jnp.int32)
    return t


def _tc_main(spikes, bmat, trialmask, msk, exp2):
    return pl.pallas_call(
        _tc_body,
        grid=(_KP // _TB, _NT_USED),
        in_specs=[
            pl.BlockSpec((1, _TB, _N_NEURONS),
                         lambda tb, tr: (_used_trial(tr), tb, 0)),
            pl.BlockSpec((_NR_PAD, _NS_PAD), lambda tb, tr: (0, 0)),
            pl.BlockSpec((1, 1, _NS_PAD), lambda tb, tr: (tr, 0, 0)),
            pl.BlockSpec((1, _NS_PAD), lambda tb, tr: (0, 0)),
            pl.BlockSpec(memory_space=pltpu.SMEM),
        ],
        out_specs=pl.BlockSpec(memory_space=pltpu.SMEM),
        out_shape=jax.ShapeDtypeStruct((1, 1), jnp.float32),
        scratch_shapes=[pltpu.VMEM((_KP, _NS_PAD), jnp.float32)],
    )(spikes, bmat, trialmask, msk, exp2)


def kernel(spikes, node_id_e, experimental_fanos_mean):
    spikes = spikes.astype(jnp.float32)
    slot = jnp.asarray(_SLOT_PAD)
    onehot = jnp.asarray(_ONEHOT)
    zrows = jnp.asarray(_ZROWS)
    bmat = _sc_membership(node_id_e.astype(jnp.int32), slot, onehot, zrows)
    exp2 = experimental_fanos_mean.astype(jnp.float32).reshape(1, len(_BINS))
    res = _tc_main(spikes, bmat, jnp.asarray(_TRIALMASK),
                   jnp.asarray(_MASK50), exp2)
    return res[0, 0]
